# Initial kernel scaffold; baseline (speedup 1.0000x reference)
import jax
import jax.numpy as jnp
from jax.experimental import pallas as pl


def kernel(x, edge_index, W1l, b1, W1r, W2l, b2, W2r):
    raise NotImplementedError("write your pallas kernel here")


# speedup vs baseline: 9.5274x; 9.5274x over previous
import jax
import jax.numpy as jnp
from jax import lax
from jax.experimental import pallas as pl
from jax.experimental.pallas import tpu as pltpu
from jax.experimental.pallas import tpu_sc as plsc

NC = 2
NS = 16
NW = NC * NS
L = 16

N = 10000
E = 320000
D = 128
H = 32

EPW = E // NW
C = 80
K = EPW // C
RPT = N // NS

_sc_mesh = plsc.VectorSubcoreMesh(core_axis_name="c", subcore_axis_name="s",
                                  num_cores=NC, num_subcores=NS)


def _make_sc_segsum(width, gather):
    def body(*refs):
        if gather:
            (tab_hbm, src_hbm, dst_hbm, zeros_hbm, psum_hbm,
             src_v, dst_v, rows_v, acc_shared, sem) = refs
        else:
            (ones_hbm, dst_hbm, zeros_hbm, psum_hbm,
             dst_v, rows_v, acc_shared, sem) = refs
        c = lax.axis_index("c")
        s = lax.axis_index("s")
        wid = c * NS + s

        if gather:
            pltpu.sync_copy(src_hbm.at[wid], src_v)
        else:
            pltpu.sync_copy(ones_hbm, rows_v)
        pltpu.sync_copy(dst_hbm.at[wid], dst_v)

        row0 = s * RPT
        pltpu.sync_copy(zeros_hbm.at[pl.ds(row0, RPT)],
                        acc_shared.at[pl.ds(row0, RPT)])
        plsc.subcore_barrier()

        def chunk(j, _):
            if gather:
                pltpu.async_copy(tab_hbm.at[src_v.at[j]], rows_v, sem).wait()
            pltpu.sync_copy(rows_v, acc_shared.at[dst_v.at[j]], add=True)
            return 0

        lax.fori_loop(0, K, chunk, 0)
        plsc.subcore_barrier()

        pltpu.sync_copy(acc_shared.at[pl.ds(row0, RPT)],
                        psum_hbm.at[c, pl.ds(row0, RPT)])

    scratch = [
        pltpu.VMEM((K, C), jnp.int32),
        pltpu.VMEM((C, width), jnp.float32),
        pltpu.VMEM_SHARED((N, width), jnp.float32),
        pltpu.SemaphoreType.DMA,
    ]
    if gather:
        scratch = [pltpu.VMEM((K, C), jnp.int32)] + scratch

    return pl.kernel(
        body,
        out_type=jax.ShapeDtypeStruct((NC, N, width), jnp.float32),
        mesh=_sc_mesh,
        scratch_types=scratch,
        compiler_params=pltpu.CompilerParams(use_tc_tiling_on_sc=False),
    )


_sc_counts = _make_sc_segsum(16, gather=False)
_sc_layer1 = _make_sc_segsum(H, gather=True)
_sc_layer2 = _make_sc_segsum(16, gather=True)

_R = 1000


def _tc_proj_kernel(x_ref, wcat_ref, y1_ref, z1_ref):
    out = jnp.dot(x_ref[...], wcat_ref[...],
                  preferred_element_type=jnp.float32)
    y1_ref[...] = out[:, :H]
    z1_ref[...] = out[:, H:]


def _tc_proj(x, wcat):
    return pl.pallas_call(
        _tc_proj_kernel,
        grid=(N // _R,),
        in_specs=[pl.BlockSpec((_R, D), lambda i: (i, 0)),
                  pl.BlockSpec((D, 2 * H), lambda i: (0, 0))],
        out_specs=[pl.BlockSpec((_R, H), lambda i: (i, 0)),
                   pl.BlockSpec((_R, H), lambda i: (i, 0))],
        out_shape=[jax.ShapeDtypeStruct((N, H), jnp.float32),
                   jax.ShapeDtypeStruct((N, H), jnp.float32)],
    )(x, wcat)


def _tc_layer1_post_kernel(pa_ref, pb_ref, ca_ref, cb_ref, z1_ref, b1_ref,
                           w2_ref, y2_ref, z2_ref, cnt_ref):
    cnt = jnp.maximum(ca_ref[...][:, 0:1] + cb_ref[...][:, 0:1], 1.0)
    mean = (pa_ref[...] + pb_ref[...]) / cnt
    h = jnp.maximum(mean + b1_ref[...] + z1_ref[...], 0.0)
    yz = jnp.dot(h, w2_ref[...], preferred_element_type=jnp.float32)
    y2_ref[...] = yz[:, :16]
    z2_ref[...] = yz[:, 16:17]
    cnt_ref[...] = cnt


def _tc_layer1_post(pa, pb, ca, cb, z1, b1, w2):
    return pl.pallas_call(
        _tc_layer1_post_kernel,
        grid=(N // _R,),
        in_specs=[pl.BlockSpec((_R, H), lambda i: (i, 0)),
                  pl.BlockSpec((_R, H), lambda i: (i, 0)),
                  pl.BlockSpec((_R, 16), lambda i: (i, 0)),
                  pl.BlockSpec((_R, 16), lambda i: (i, 0)),
                  pl.BlockSpec((_R, H), lambda i: (i, 0)),
                  pl.BlockSpec((1, H), lambda i: (0, 0)),
                  pl.BlockSpec((H, 17), lambda i: (0, 0))],
        out_specs=[pl.BlockSpec((_R, 16), lambda i: (i, 0)),
                   pl.BlockSpec((_R, 1), lambda i: (i, 0)),
                   pl.BlockSpec((_R, 1), lambda i: (i, 0))],
        out_shape=[jax.ShapeDtypeStruct((N, 16), jnp.float32),
                   jax.ShapeDtypeStruct((N, 1), jnp.float32),
                   jax.ShapeDtypeStruct((N, 1), jnp.float32)],
    )(pa, pb, ca, cb, z1, b1, w2)


def _tc_final_kernel(qa_ref, qb_ref, cnt_ref, z2_ref, b2_ref, out_ref):
    s = qa_ref[...][:, 0:1] + qb_ref[...][:, 0:1]
    out_ref[...] = s / cnt_ref[...] + b2_ref[...] + z2_ref[...]


def _tc_final(qa, qb, cnt, z2, b2):
    return pl.pallas_call(
        _tc_final_kernel,
        grid=(N // _R,),
        in_specs=[pl.BlockSpec((_R, 16), lambda i: (i, 0)),
                  pl.BlockSpec((_R, 16), lambda i: (i, 0)),
                  pl.BlockSpec((_R, 1), lambda i: (i, 0)),
                  pl.BlockSpec((_R, 1), lambda i: (i, 0)),
                  pl.BlockSpec((1, 1), lambda i: (0, 0))],
        out_specs=pl.BlockSpec((_R, 1), lambda i: (i, 0)),
        out_shape=jax.ShapeDtypeStruct((N, 1), jnp.float32),
    )(qa, qb, cnt, z2, b2)


def kernel(x, edge_index, W1l, b1, W1r, W2l, b2, W2r):
    src = edge_index[0].astype(jnp.int32).reshape(NW, K, C)
    dst = edge_index[1].astype(jnp.int32).reshape(NW, K, C)

    zeros16 = jnp.zeros((N, 16), jnp.float32)
    zeros32 = jnp.zeros((N, H), jnp.float32)
    e0rows = jnp.zeros((C, 16), jnp.float32).at[:, 0].set(1.0)

    cnts = _sc_counts(e0rows, dst, zeros16)
    wcat = jnp.concatenate([W1l.T, W1r.T], axis=1)
    y1, z1 = _tc_proj(x, wcat)

    psum = _sc_layer1(y1, src, dst, zeros32)

    w2 = jnp.zeros((H, 17), jnp.float32)
    w2 = w2.at[:, 0].set(W2l[0]).at[:, 16].set(W2r[0])
    y2, z2, cnt = _tc_layer1_post(psum[0], psum[1], cnts[0], cnts[1],
                                  z1, b1.reshape(1, H), w2)

    q = _sc_layer2(y2, src, dst, zeros16)

    out = _tc_final(q[0], q[1], cnt, z2, b2.reshape(1, 1))
    return out


# speedup vs baseline: 14.8039x; 1.5538x over previous
import jax
import jax.numpy as jnp
from jax import lax
from jax.experimental import pallas as pl
from jax.experimental.pallas import tpu as pltpu
from jax.experimental.pallas import tpu_sc as plsc

NC = 2
NS = 16
NW = NC * NS
L = 16

N = 10000
E = 320000
D = 128
H = 32

EPW = E // NW
C = 125
K = EPW // C
KH = K // 2
G = 8
RPT = N // NS

_sc_mesh = plsc.VectorSubcoreMesh(core_axis_name="c", subcore_axis_name="s",
                                  num_cores=NC, num_subcores=NS)


def _make_sc_segsum(width, gather):
    def body(*refs):
        if gather:
            (tab_hbm, src_hbm, dst_hbm, zeros_hbm, psum_hbm,
             src_v, dst_v, buf0, buf1, acc_shared, gsem0, gsem1) = refs
        else:
            (ones_hbm, dst_hbm, zeros_hbm, psum_hbm,
             dst_v, rows_v, acc_shared, csem) = refs
        c = lax.axis_index("c")
        s = lax.axis_index("s")
        wid = c * NS + s

        if gather:
            pltpu.sync_copy(src_hbm.at[wid], src_v)
        else:
            pltpu.sync_copy(ones_hbm, rows_v)
        pltpu.sync_copy(dst_hbm.at[wid], dst_v)

        row0 = s * RPT
        pltpu.sync_copy(zeros_hbm.at[pl.ds(row0, RPT)],
                        acc_shared.at[pl.ds(row0, RPT)])
        plsc.subcore_barrier()

        if gather:
            pltpu.async_copy(tab_hbm.at[src_v.at[0]], buf0, gsem0)

            def pair(jj, _):
                j0 = 2 * jj
                j1 = j0 + 1
                pltpu.async_copy(tab_hbm.at[src_v.at[j1]], buf1, gsem1)
                pltpu.make_async_copy(tab_hbm.at[src_v.at[j0]],
                                      buf0, gsem0).wait()
                pltpu.sync_copy(buf0, acc_shared.at[dst_v.at[j0]], add=True)

                @pl.when(jj + 1 < KH)
                def _():
                    pltpu.async_copy(tab_hbm.at[src_v.at[j0 + 2]],
                                     buf0, gsem0)

                pltpu.make_async_copy(tab_hbm.at[src_v.at[j1]],
                                      buf1, gsem1).wait()
                pltpu.sync_copy(buf1, acc_shared.at[dst_v.at[j1]], add=True)
                return 0

            lax.fori_loop(0, KH, pair, 0)
        else:
            def grp(g, _):
                for t in range(G):
                    pltpu.async_copy(rows_v,
                                     acc_shared.at[dst_v.at[g * G + t]],
                                     csem, add=True)
                for t in range(G):
                    pltpu.make_async_copy(
                        rows_v, acc_shared.at[dst_v.at[g * G + t]],
                        csem).wait()
                return 0

            lax.fori_loop(0, K // G, grp, 0)

        plsc.subcore_barrier()

        pltpu.sync_copy(acc_shared.at[pl.ds(row0, RPT)],
                        psum_hbm.at[c, pl.ds(row0, RPT)])

    if gather:
        scratch = [
            pltpu.VMEM((K, C), jnp.int32),
            pltpu.VMEM((K, C), jnp.int32),
            pltpu.VMEM((C, width), jnp.float32),
            pltpu.VMEM((C, width), jnp.float32),
            pltpu.VMEM_SHARED((N, width), jnp.float32),
            pltpu.SemaphoreType.DMA,
            pltpu.SemaphoreType.DMA,
        ]
    else:
        scratch = [
            pltpu.VMEM((K, C), jnp.int32),
            pltpu.VMEM((C, width), jnp.float32),
            pltpu.VMEM_SHARED((N, width), jnp.float32),
            pltpu.SemaphoreType.DMA,
        ]

    return pl.kernel(
        body,
        out_type=jax.ShapeDtypeStruct((NC, N, width), jnp.float32),
        mesh=_sc_mesh,
        scratch_types=scratch,
        compiler_params=pltpu.CompilerParams(use_tc_tiling_on_sc=False),
    )


_sc_counts = _make_sc_segsum(16, gather=False)
_sc_layer1 = _make_sc_segsum(H, gather=True)
_sc_layer2 = _make_sc_segsum(16, gather=True)

_R = 1000


def _tc_proj_kernel(x_ref, wcat_ref, y1_ref, z1_ref):
    out = jnp.dot(x_ref[...], wcat_ref[...],
                  preferred_element_type=jnp.float32)
    y1_ref[...] = out[:, :H]
    z1_ref[...] = out[:, H:]


def _tc_proj(x, wcat):
    return pl.pallas_call(
        _tc_proj_kernel,
        grid=(N // _R,),
        in_specs=[pl.BlockSpec((_R, D), lambda i: (i, 0)),
                  pl.BlockSpec((D, 2 * H), lambda i: (0, 0))],
        out_specs=[pl.BlockSpec((_R, H), lambda i: (i, 0)),
                   pl.BlockSpec((_R, H), lambda i: (i, 0))],
        out_shape=[jax.ShapeDtypeStruct((N, H), jnp.float32),
                   jax.ShapeDtypeStruct((N, H), jnp.float32)],
    )(x, wcat)


def _tc_layer1_post_kernel(pa_ref, pb_ref, ca_ref, cb_ref, z1_ref, b1_ref,
                           w2_ref, y2_ref, z2_ref, cnt_ref):
    cnt = jnp.maximum(ca_ref[...][:, 0:1] + cb_ref[...][:, 0:1], 1.0)
    mean = (pa_ref[...] + pb_ref[...]) / cnt
    h = jnp.maximum(mean + b1_ref[...] + z1_ref[...], 0.0)
    yz = jnp.dot(h, w2_ref[...], preferred_element_type=jnp.float32)
    y2_ref[...] = yz[:, :16]
    z2_ref[...] = yz[:, 16:17]
    cnt_ref[...] = cnt


def _tc_layer1_post(pa, pb, ca, cb, z1, b1, w2):
    return pl.pallas_call(
        _tc_layer1_post_kernel,
        grid=(N // _R,),
        in_specs=[pl.BlockSpec((_R, H), lambda i: (i, 0)),
                  pl.BlockSpec((_R, H), lambda i: (i, 0)),
                  pl.BlockSpec((_R, 16), lambda i: (i, 0)),
                  pl.BlockSpec((_R, 16), lambda i: (i, 0)),
                  pl.BlockSpec((_R, H), lambda i: (i, 0)),
                  pl.BlockSpec((1, H), lambda i: (0, 0)),
                  pl.BlockSpec((H, 17), lambda i: (0, 0))],
        out_specs=[pl.BlockSpec((_R, 16), lambda i: (i, 0)),
                   pl.BlockSpec((_R, 1), lambda i: (i, 0)),
                   pl.BlockSpec((_R, 1), lambda i: (i, 0))],
        out_shape=[jax.ShapeDtypeStruct((N, 16), jnp.float32),
                   jax.ShapeDtypeStruct((N, 1), jnp.float32),
                   jax.ShapeDtypeStruct((N, 1), jnp.float32)],
    )(pa, pb, ca, cb, z1, b1, w2)


def _tc_final_kernel(qa_ref, qb_ref, cnt_ref, z2_ref, b2_ref, out_ref):
    s = qa_ref[...][:, 0:1] + qb_ref[...][:, 0:1]
    out_ref[...] = s / cnt_ref[...] + b2_ref[...] + z2_ref[...]


def _tc_final(qa, qb, cnt, z2, b2):
    return pl.pallas_call(
        _tc_final_kernel,
        grid=(N // _R,),
        in_specs=[pl.BlockSpec((_R, 16), lambda i: (i, 0)),
                  pl.BlockSpec((_R, 16), lambda i: (i, 0)),
                  pl.BlockSpec((_R, 1), lambda i: (i, 0)),
                  pl.BlockSpec((_R, 1), lambda i: (i, 0)),
                  pl.BlockSpec((1, 1), lambda i: (0, 0))],
        out_specs=pl.BlockSpec((_R, 1), lambda i: (i, 0)),
        out_shape=jax.ShapeDtypeStruct((N, 1), jnp.float32),
    )(qa, qb, cnt, z2, b2)


def kernel(x, edge_index, W1l, b1, W1r, W2l, b2, W2r):
    src = edge_index[0].astype(jnp.int32).reshape(NW, K, C)
    dst = edge_index[1].astype(jnp.int32).reshape(NW, K, C)

    zeros16 = jnp.zeros((N, 16), jnp.float32)
    zeros32 = jnp.zeros((N, H), jnp.float32)
    e0rows = jnp.zeros((C, 16), jnp.float32).at[:, 0].set(1.0)

    cnts = _sc_counts(e0rows, dst, zeros16)
    wcat = jnp.concatenate([W1l.T, W1r.T], axis=1)
    y1, z1 = _tc_proj(x, wcat)

    psum = _sc_layer1(y1, src, dst, zeros32)

    w2 = jnp.zeros((H, 17), jnp.float32)
    w2 = w2.at[:, 0].set(W2l[0]).at[:, 16].set(W2r[0])
    y2, z2, cnt = _tc_layer1_post(psum[0], psum[1], cnts[0], cnts[1],
                                  z1, b1.reshape(1, H), w2)

    q = _sc_layer2(y2, src, dst, zeros16)

    out = _tc_final(q[0], q[1], cnt, z2, b2.reshape(1, 1))
    return out


# speedup vs baseline: 15.3909x; 1.0396x over previous
import jax
import jax.numpy as jnp
from jax import lax
from jax.experimental import pallas as pl
from jax.experimental.pallas import tpu as pltpu
from jax.experimental.pallas import tpu_sc as plsc

NC = 2
NS = 16
NW = NC * NS

N = 10000
NP = 10240
E = 320000
D = 128
H = 32

EPW = E // NW
C = 125
K = EPW // C
KH = K // 2
G = 8
RPT = NP // NS

_sc_mesh = plsc.VectorSubcoreMesh(core_axis_name="c", subcore_axis_name="s",
                                  num_cores=NC, num_subcores=NS)


def _make_sc_segsum(width, gather):
    wsuf = (width,) if width > 1 else ()

    def body(*refs):
        if gather:
            (tab_hbm, src_hbm, dst_hbm, zeros_hbm, psum_hbm,
             src_v, dst_v, buf0, buf1, acc_shared, gsem0, gsem1) = refs
        else:
            (ones_hbm, dst_hbm, zeros_hbm, psum_hbm,
             dst_v, rows_v, acc_shared, csem) = refs
        c = lax.axis_index("c")
        s = lax.axis_index("s")
        wid = c * NS + s

        if gather:
            pltpu.sync_copy(src_hbm.at[wid], src_v)
        else:
            pltpu.sync_copy(ones_hbm, rows_v)
        pltpu.sync_copy(dst_hbm.at[wid], dst_v)

        row0 = s * RPT
        pltpu.sync_copy(zeros_hbm.at[pl.ds(row0, RPT)],
                        acc_shared.at[pl.ds(row0, RPT)])
        plsc.subcore_barrier()

        if gather:
            pltpu.async_copy(tab_hbm.at[src_v.at[0]], buf0, gsem0)

            def pair(jj, _):
                j0 = 2 * jj
                j1 = j0 + 1
                pltpu.async_copy(tab_hbm.at[src_v.at[j1]], buf1, gsem1)
                pltpu.make_async_copy(tab_hbm.at[src_v.at[j0]],
                                      buf0, gsem0).wait()
                pltpu.sync_copy(buf0, acc_shared.at[dst_v.at[j0]], add=True)

                @pl.when(jj + 1 < KH)
                def _():
                    pltpu.async_copy(tab_hbm.at[src_v.at[j0 + 2]],
                                     buf0, gsem0)

                pltpu.make_async_copy(tab_hbm.at[src_v.at[j1]],
                                      buf1, gsem1).wait()
                pltpu.sync_copy(buf1, acc_shared.at[dst_v.at[j1]], add=True)
                return 0

            lax.fori_loop(0, KH, pair, 0)
        else:
            def grp(g, _):
                for t in range(G):
                    pltpu.async_copy(rows_v,
                                     acc_shared.at[dst_v.at[g * G + t]],
                                     csem, add=True)
                for t in range(G):
                    pltpu.make_async_copy(
                        rows_v, acc_shared.at[dst_v.at[g * G + t]],
                        csem).wait()
                return 0

            lax.fori_loop(0, K // G, grp, 0)

        plsc.subcore_barrier()

        pltpu.sync_copy(acc_shared.at[pl.ds(row0, RPT)],
                        psum_hbm.at[c, pl.ds(row0, RPT)])

    if gather:
        scratch = [
            pltpu.VMEM((K, C), jnp.int32),
            pltpu.VMEM((K, C), jnp.int32),
            pltpu.VMEM((C,) + wsuf, jnp.float32),
            pltpu.VMEM((C,) + wsuf, jnp.float32),
            pltpu.VMEM_SHARED((NP,) + wsuf, jnp.float32),
            pltpu.SemaphoreType.DMA,
            pltpu.SemaphoreType.DMA,
        ]
    else:
        scratch = [
            pltpu.VMEM((K, C), jnp.int32),
            pltpu.VMEM((C,) + wsuf, jnp.float32),
            pltpu.VMEM_SHARED((NP,) + wsuf, jnp.float32),
            pltpu.SemaphoreType.DMA,
        ]

    return pl.kernel(
        body,
        out_type=jax.ShapeDtypeStruct((NC, NP) + wsuf, jnp.float32),
        mesh=_sc_mesh,
        scratch_types=scratch,
        compiler_params=pltpu.CompilerParams(use_tc_tiling_on_sc=False),
    )


_sc_counts = _make_sc_segsum(1, gather=False)
_sc_layer1 = _make_sc_segsum(H, gather=True)
_sc_layer2 = _make_sc_segsum(1, gather=True)

_R = 1024


def _tc_proj_kernel(x_ref, wcat_ref, y1_ref, z1_ref):
    out = jnp.dot(x_ref[...], wcat_ref[...],
                  preferred_element_type=jnp.float32)
    y1_ref[...] = out[:, :H]
    z1_ref[...] = out[:, H:]


def _tc_proj(x, wcat):
    return pl.pallas_call(
        _tc_proj_kernel,
        grid=(NP // _R,),
        in_specs=[pl.BlockSpec((_R, D), lambda i: (i, 0)),
                  pl.BlockSpec((D, 2 * H), lambda i: (0, 0))],
        out_specs=[pl.BlockSpec((_R, H), lambda i: (i, 0)),
                   pl.BlockSpec((_R, H), lambda i: (i, 0))],
        out_shape=[jax.ShapeDtypeStruct((NP, H), jnp.float32),
                   jax.ShapeDtypeStruct((NP, H), jnp.float32)],
    )(x, wcat)


def _tc_layer1_post_kernel(pa_ref, pb_ref, ca_ref, cb_ref, z1_ref, b1_ref,
                           w2_ref, y2_ref, z2_ref, cnt_ref):
    cnt = jnp.maximum(ca_ref[...] + cb_ref[...], 1.0)
    mean = (pa_ref[...] + pb_ref[...]) / cnt
    h = jnp.maximum(mean + b1_ref[...] + z1_ref[...], 0.0)
    yz = jnp.dot(h, w2_ref[...], preferred_element_type=jnp.float32)
    y2_ref[...] = yz[:, 0:1]
    z2_ref[...] = yz[:, 1:2]
    cnt_ref[...] = cnt


def _tc_layer1_post(pa, pb, ca, cb, z1, b1, w2):
    return pl.pallas_call(
        _tc_layer1_post_kernel,
        grid=(NP // _R,),
        in_specs=[pl.BlockSpec((_R, H), lambda i: (i, 0)),
                  pl.BlockSpec((_R, H), lambda i: (i, 0)),
                  pl.BlockSpec((_R, 1), lambda i: (i, 0)),
                  pl.BlockSpec((_R, 1), lambda i: (i, 0)),
                  pl.BlockSpec((_R, H), lambda i: (i, 0)),
                  pl.BlockSpec((1, H), lambda i: (0, 0)),
                  pl.BlockSpec((H, 2), lambda i: (0, 0))],
        out_specs=[pl.BlockSpec((_R, 1), lambda i: (i, 0)),
                   pl.BlockSpec((_R, 1), lambda i: (i, 0)),
                   pl.BlockSpec((_R, 1), lambda i: (i, 0))],
        out_shape=[jax.ShapeDtypeStruct((NP, 1), jnp.float32),
                   jax.ShapeDtypeStruct((NP, 1), jnp.float32),
                   jax.ShapeDtypeStruct((NP, 1), jnp.float32)],
    )(pa, pb, ca, cb, z1, b1, w2)


def _tc_final_kernel(qa_ref, qb_ref, cnt_ref, z2_ref, b2_ref, out_ref):
    s = qa_ref[...] + qb_ref[...]
    out_ref[...] = s / cnt_ref[...] + b2_ref[...] + z2_ref[...]


def _tc_final(qa, qb, cnt, z2, b2):
    return pl.pallas_call(
        _tc_final_kernel,
        grid=(NP // _R,),
        in_specs=[pl.BlockSpec((_R, 1), lambda i: (i, 0)),
                  pl.BlockSpec((_R, 1), lambda i: (i, 0)),
                  pl.BlockSpec((_R, 1), lambda i: (i, 0)),
                  pl.BlockSpec((_R, 1), lambda i: (i, 0)),
                  pl.BlockSpec((1, 1), lambda i: (0, 0))],
        out_specs=pl.BlockSpec((_R, 1), lambda i: (i, 0)),
        out_shape=jax.ShapeDtypeStruct((NP, 1), jnp.float32),
    )(qa, qb, cnt, z2, b2)


def kernel(x, edge_index, W1l, b1, W1r, W2l, b2, W2r):
    src = edge_index[0].astype(jnp.int32).reshape(NW, K, C)
    dst = edge_index[1].astype(jnp.int32).reshape(NW, K, C)

    zeros1 = jnp.zeros((NP,), jnp.float32)
    zeros32 = jnp.zeros((NP, H), jnp.float32)
    ones_c = jnp.ones((C,), jnp.float32)

    cnts = _sc_counts(ones_c, dst, zeros1)
    wcat = jnp.concatenate([W1l.T, W1r.T], axis=1)
    y1, z1 = _tc_proj(x, wcat)

    psum = _sc_layer1(y1, src, dst, zeros32)

    w2 = jnp.stack([W2l[0], W2r[0]], axis=1)
    y2, z2, cnt = _tc_layer1_post(psum[0], psum[1],
                                  cnts[0].reshape(NP, 1),
                                  cnts[1].reshape(NP, 1),
                                  z1, b1.reshape(1, H), w2)

    q = _sc_layer2(y2.reshape(NP), src, dst, zeros1)

    out = _tc_final(q[0].reshape(NP, 1), q[1].reshape(NP, 1),
                    cnt, z2, b2.reshape(1, 1))
    return out[:N]
